# baseline (speedup 1.0000x reference)
import functools

import jax
import jax.numpy as jnp
from jax import lax
from jax.experimental import pallas as pl
from jax.experimental.pallas import tpu as pltpu
from jax.experimental.pallas import tpu_sc as plsc

D_MODEL = 1024
NUM_EMB = 4

_NC, _NS = 2, 16
_NW = _NC * _NS


@functools.lru_cache(maxsize=None)
def _make_lookup(B: int, D: int, C: int):
    assert B % (8 * _NW) == 0
    b_per_w = B // _NW
    assert b_per_w % C == 0
    n_chunks = b_per_w // C
    mesh = plsc.VectorSubcoreMesh(core_axis_name="c", subcore_axis_name="s")

    @functools.partial(
        pl.kernel,
        mesh=mesh,
        out_type=jax.ShapeDtypeStruct((B, D), jnp.float32),
        scratch_types=[
            pltpu.VMEM((b_per_w,), jnp.int32),
            pltpu.VMEM((2, C, D), jnp.float32),
            pltpu.SemaphoreType.DMA,
            pltpu.SemaphoreType.DMA,
            pltpu.SemaphoreType.DMA,
            pltpu.SemaphoreType.DMA,
        ],
    )
    def lookup(table_hbm, idx_hbm, out_hbm, idx_v, rows_v, g0, g1, w0, w1):
        gsem = (g0, g1)
        wsem = (w0, w1)
        wid = lax.axis_index("s") * _NC + lax.axis_index("c")
        base = wid * b_per_w
        pltpu.sync_copy(idx_hbm.at[pl.ds(base, b_per_w)], idx_v)

        def gather(i):
            b = i % 2
            return pltpu.async_copy(
                table_hbm.at[idx_v.at[pl.ds(i * C, C)]], rows_v.at[b], gsem[b]
            )

        gathers = {0: gather(0)}
        writes = {}
        for i in range(n_chunks):
            b = i % 2
            if i - 1 in writes:
                writes.pop(i - 1).wait()
            if i + 1 < n_chunks:
                gathers[i + 1] = gather(i + 1)
            gathers.pop(i).wait()
            writes[i] = pltpu.async_copy(
                rows_v.at[b], out_hbm.at[pl.ds(base + i * C, C)], wsem[b]
            )
        writes.pop(n_chunks - 1).wait()

    return lookup


def kernel(class_ids, W):
    ids = class_ids.reshape(-1).astype(jnp.int32)
    out = _make_lookup(ids.shape[0], W.shape[1], 32)(W, ids)
    return out.reshape(class_ids.shape + (W.shape[1],))


# speedup vs baseline: 1.0009x; 1.0009x over previous
import functools

import jax
import jax.numpy as jnp
from jax import lax
from jax.experimental import pallas as pl
from jax.experimental.pallas import tpu as pltpu
from jax.experimental.pallas import tpu_sc as plsc

D_MODEL = 1024
NUM_EMB = 4

_NC, _NS = 2, 16
_NW = _NC * _NS


@functools.lru_cache(maxsize=None)
def _make_lookup(B: int, D: int, C: int, n_buf: int):
    assert B % (8 * _NW) == 0
    b_per_w = B // _NW
    assert b_per_w % C == 0
    n_chunks = b_per_w // C
    mesh = plsc.VectorSubcoreMesh(core_axis_name="c", subcore_axis_name="s")

    @functools.partial(
        pl.kernel,
        mesh=mesh,
        out_type=jax.ShapeDtypeStruct((B, D), jnp.float32),
        scratch_types=[
            pltpu.VMEM((b_per_w,), jnp.int32),
            pltpu.VMEM((n_buf, C, D), jnp.float32),
        ]
        + [pltpu.SemaphoreType.DMA] * (2 * n_buf),
    )
    def lookup(table_hbm, idx_hbm, out_hbm, idx_v, rows_v, *sems):
        gsem = sems[:n_buf]
        wsem = sems[n_buf:]
        wid = lax.axis_index("s") * _NC + lax.axis_index("c")
        base = wid * b_per_w
        pltpu.sync_copy(idx_hbm.at[pl.ds(base, b_per_w)], idx_v)

        def gather(i):
            b = i % n_buf
            return pltpu.async_copy(
                table_hbm.at[idx_v.at[pl.ds(i * C, C)]], rows_v.at[b], gsem[b]
            )

        gathers = {0: gather(0)}
        writes = {}
        for i in range(n_chunks):
            b = i % n_buf
            gathers.pop(i).wait()
            writes[i] = pltpu.async_copy(
                rows_v.at[b], out_hbm.at[pl.ds(base + i * C, C)], wsem[b]
            )
            if i + 1 < n_chunks:
                j = i + 1 - n_buf
                if j in writes:
                    writes.pop(j).wait()
                gathers[i + 1] = gather(i + 1)
        for i in sorted(writes):
            writes.pop(i).wait()

    return lookup


def kernel(class_ids, W):
    ids = class_ids.reshape(-1).astype(jnp.int32)
    out = _make_lookup(ids.shape[0], W.shape[1], 64, 1)(W, ids)
    return out.reshape(class_ids.shape + (W.shape[1],))


# speedup vs baseline: 1.2591x; 1.2580x over previous
import functools

import jax
import jax.numpy as jnp
from jax import lax
from jax.experimental import pallas as pl
from jax.experimental.pallas import tpu as pltpu
from jax.experimental.pallas import tpu_sc as plsc

D_MODEL = 1024
NUM_EMB = 4

_NC, _NS = 2, 16
_NW = _NC * _NS
_G = 16


@functools.lru_cache(maxsize=None)
def _make_lookup(B: int, D: int):
    assert B % (_G * _NW) == 0
    b_per_w = B // _NW
    n_groups = b_per_w // _G
    n_col = D // 16
    mesh = plsc.VectorSubcoreMesh(core_axis_name="c", subcore_axis_name="s")

    @functools.partial(
        pl.kernel,
        mesh=mesh,
        out_type=jax.ShapeDtypeStruct((B * D,), jnp.float32),
        scratch_types=[
            pltpu.VMEM((NUM_EMB * D,), jnp.float32),
            pltpu.VMEM((b_per_w,), jnp.int32),
            pltpu.VMEM((2 * _G * D,), jnp.float32),
            pltpu.SemaphoreType.DMA,
            pltpu.SemaphoreType.DMA,
        ],
    )
    def lookup(table_hbm, idx_hbm, out_hbm, table_v, idx_v, buf_v, w0, w1):
        wsem = (w0, w1)
        wid = lax.axis_index("s") * _NC + lax.axis_index("c")
        base = wid * b_per_w
        pltpu.sync_copy(table_hbm, table_v)
        pltpu.sync_copy(idx_hbm.at[pl.ds(base, b_per_w)], idx_v)

        def group(g, carry):
            b = g % 2
            bbase = b * (_G * D)

            for pb in range(2):
                @pl.when((g >= 2) & (b == pb))
                def _():
                    pltpu.make_async_copy(
                        buf_v.at[pl.ds(pb * (_G * D), _G * D)],
                        out_hbm.at[pl.ds(base * D, _G * D)],
                        wsem[pb],
                    ).wait()

            ids16 = idx_v[pl.ds(g * _G, _G)]
            for r in range(_G):
                roff = bbase + r * D
                coff = ids16[r] * D
                for k in range(n_col):
                    buf_v[pl.ds(roff + 16 * k, 16)] = table_v[
                        pl.ds(coff + 16 * k, 16)
                    ]

            for pb in range(2):
                @pl.when(b == pb)
                def _():
                    pltpu.async_copy(
                        buf_v.at[pl.ds(pb * (_G * D), _G * D)],
                        out_hbm.at[pl.ds((base + g * _G) * D, _G * D)],
                        wsem[pb],
                    )

            return carry

        lax.fori_loop(0, n_groups, group, 0)
        for pb in range(2):
            pltpu.make_async_copy(
                buf_v.at[pl.ds(pb * (_G * D), _G * D)],
                out_hbm.at[pl.ds(base * D, _G * D)],
                wsem[pb],
            ).wait()

    return lookup


def kernel(class_ids, W):
    ids = class_ids.reshape(-1).astype(jnp.int32)
    out = _make_lookup(ids.shape[0], W.shape[1])(W.reshape(-1), ids)
    return out.reshape(class_ids.shape + (W.shape[1],))


# speedup vs baseline: 2.7694x; 2.1994x over previous
import functools

import jax
import jax.numpy as jnp
from jax import lax
from jax.experimental import pallas as pl
from jax.experimental.pallas import tpu as pltpu
from jax.experimental.pallas import tpu_sc as plsc

D_MODEL = 1024
NUM_EMB = 4

_NC, _NS = 2, 16
_NW = _NC * _NS
_G = 16


@functools.lru_cache(maxsize=None)
def _make_lookup(B: int, D: int):
    assert B % (_G * _NW) == 0
    b_per_w = B // _NW
    n_groups = b_per_w // _G
    mesh = plsc.VectorSubcoreMesh(core_axis_name="c", subcore_axis_name="s")

    @functools.partial(
        pl.kernel,
        mesh=mesh,
        out_type=jax.ShapeDtypeStruct((B * D,), jnp.float32),
        scratch_types=[
            pltpu.VMEM((NUM_EMB * D,), jnp.float32),
            pltpu.VMEM((b_per_w,), jnp.int32),
            pltpu.VMEM((16 * D,), jnp.float32),
            pltpu.SemaphoreType.DMA,
        ],
    )
    def lookup(table_hbm, idx_hbm, out_hbm, table_v, idx_v, fake_v, wsem):
        wid = lax.axis_index("s") * _NC + lax.axis_index("c")
        base = wid * b_per_w
        pltpu.sync_copy(table_hbm, table_v)
        pltpu.sync_copy(idx_hbm.at[pl.ds(base, b_per_w)], idx_v)

        def group(g, carry):
            ids16 = idx_v[pl.ds(g * _G, _G)]
            row0 = base + g * _G
            for r in range(_G):
                pltpu.async_copy(
                    table_v.at[pl.ds(ids16[r] * D, D)],
                    out_hbm.at[pl.ds((row0 + r) * D, D)],
                    wsem,
                )
            return carry

        lax.fori_loop(0, n_groups, group, 0)

        def drain(i, carry):
            pltpu.make_async_copy(
                out_hbm.at[pl.ds(base * D, 16 * D)], fake_v, wsem
            ).wait()
            return carry

        lax.fori_loop(0, b_per_w // 16, drain, 0)

    return lookup


def kernel(class_ids, W):
    ids = class_ids.reshape(-1).astype(jnp.int32)
    out = _make_lookup(ids.shape[0], W.shape[1])(W.reshape(-1), ids)
    return out.reshape(class_ids.shape + (W.shape[1],))


# speedup vs baseline: 2.7762x; 1.0024x over previous
import functools

import jax
import jax.numpy as jnp
from jax import lax
from jax.experimental import pallas as pl
from jax.experimental.pallas import tpu as pltpu
from jax.experimental.pallas import tpu_sc as plsc

D_MODEL = 1024
NUM_EMB = 4

_NC, _NS = 2, 16
_NW = _NC * _NS
_G = 16


@functools.lru_cache(maxsize=None)
def _make_lookup(B: int, D: int, V: int):
    assert B % (_G * _NW) == 0
    b_per_w = B // _NW
    n_groups = b_per_w // _G
    mesh = plsc.VectorSubcoreMesh(core_axis_name="c", subcore_axis_name="s")

    @functools.partial(
        pl.kernel,
        mesh=mesh,
        out_type=jax.ShapeDtypeStruct((B * D,), jnp.float32),
        scratch_types=[
            pltpu.VMEM((V * D,), jnp.float32),
            pltpu.VMEM((b_per_w,), jnp.int32),
            pltpu.VMEM((64 * D,), jnp.float32),
            pltpu.SemaphoreType.DMA,
        ],
    )
    def lookup(table_hbm, idx_hbm, out_hbm, table_v, idx_v, fake_v, wsem):
        wid = lax.axis_index("s") * _NC + lax.axis_index("c")
        base = wid * b_per_w
        pltpu.sync_copy(table_hbm, table_v)
        pltpu.sync_copy(idx_hbm.at[pl.ds(base, b_per_w)], idx_v)

        def group(g, carry):
            ids16 = idx_v[pl.ds(g * _G, _G)]
            row0 = base + g * _G
            for r in range(_G):
                pltpu.async_copy(
                    table_v.at[pl.ds(ids16[r] * D, D)],
                    out_hbm.at[pl.ds((row0 + r) * D, D)],
                    wsem,
                )
            return carry

        lax.fori_loop(0, n_groups, group, 0)

        def drain(i, carry):
            pltpu.make_async_copy(
                out_hbm.at[pl.ds(base * D, 64 * D)], fake_v, wsem
            ).wait()
            return carry

        lax.fori_loop(0, b_per_w // 64, drain, 0)

    return lookup


def kernel(class_ids, W):
    ids = class_ids.reshape(-1).astype(jnp.int32)
    out = _make_lookup(ids.shape[0], W.shape[1], W.shape[0])(W.reshape(-1), ids)
    return out.reshape(class_ids.shape + (W.shape[1],))
